# baseline (speedup 1.0000x reference)
import functools

import jax
import jax.numpy as jnp
from jax import lax
from jax.experimental import pallas as pl
from jax.experimental.pallas import tpu as pltpu
from jax.experimental.pallas import tpu_sc as plsc

_BATCH = 16384
_D = 64
_NC = 2
_NS = 16
_NW = _NC * _NS
_BPW = _BATCH // _NW
_CH = 128
_NCH = _BPW // _CH


def _gather_body(idx_hbm, ut_hbm, it_hbm, u_hbm, p_hbm, n_hbm,
                 idx_v, stage_v, sem):
    wid = lax.axis_index("s") * _NC + lax.axis_index("c")
    base = wid * _BPW
    pltpu.sync_copy(idx_hbm.at[wid], idx_v)

    tables = (ut_hbm, it_hbm, it_hbm)
    outs = (u_hbm, p_hbm, n_hbm)

    for c in range(3):
        copies = []
        for j in range(_NCH):
            copies.append(
                pltpu.async_copy(
                    tables[c].at[idx_v.at[c, j]],
                    stage_v.at[pl.ds(j * _CH, _CH)],
                    sem,
                )
            )
        for cp in copies:
            cp.wait()
        pltpu.sync_copy(stage_v, outs[c].at[pl.ds(base, _BPW)])


_mesh = plsc.VectorSubcoreMesh(core_axis_name="c", subcore_axis_name="s")

_gather_call = functools.partial(
    pl.kernel,
    out_type=[jax.ShapeDtypeStruct((_BATCH, 2 * _D), jnp.float32)] * 3,
    mesh=_mesh,
    scratch_types=[
        pltpu.VMEM((3, _NCH, _CH), jnp.int32),
        pltpu.VMEM((_BPW, 2 * _D), jnp.float32),
        pltpu.SemaphoreType.DMA,
    ],
    compiler_params=pltpu.CompilerParams(use_tc_tiling_on_sc=True),
)(_gather_body)


def kernel(user, pos, neg, user_table, item_table):
    idx = jnp.stack(
        [
            (user >> 1).reshape(_NW, _NCH, _CH),
            (pos >> 1).reshape(_NW, _NCH, _CH),
            (neg >> 1).reshape(_NW, _NCH, _CH),
        ],
        axis=1,
    )
    u2 = user_table.reshape(-1, 2 * _D)
    i2 = item_table.reshape(-1, 2 * _D)
    u_f, p_f, n_f = _gather_call(idx, u2, i2)

    def _half(full, v):
        odd = (v & 1) == 1
        return jnp.where(odd, full[:, _D:], full[:, :_D])

    return jnp.concatenate(
        [_half(u_f, user), _half(p_f, pos), _half(n_f, neg)], axis=-1
    ).reshape(_BATCH, 1, 3 * _D)
